# baseline (speedup 1.0000x reference)
import functools

import jax
import jax.numpy as jnp
from jax.experimental import pallas as pl
from jax.experimental.pallas import tpu as pltpu

NUM_EXPERTS = 8
D_MODEL = 768
HIDDEN = 3072
SEQ = 2048

TBLK = 256
H2 = 2
HBLK = HIDDEN // H2
NEG = -1e30


def _moe_body(x_ref, wg_ref, bg_ref, w1_ref, b1_ref, w2_ref, b2_ref, o_ref,
              s0_ref, s1_ref, i0_ref, i1_ref):
    e = pl.program_id(1)
    h = pl.program_id(2)

    @pl.when((e == 0) & (h == 0))
    def _router():
        xb = x_ref[...]
        logits = jnp.dot(xb, wg_ref[...], preferred_element_type=jnp.float32)
        logits = logits + bg_ref[...]
        lane = jax.lax.broadcasted_iota(jnp.int32, logits.shape, 1)
        m0 = jnp.max(logits, axis=1, keepdims=True)
        i0 = jnp.min(jnp.where(logits >= m0, lane, 2**30), axis=1, keepdims=True)
        masked = jnp.where(lane == i0, NEG, logits)
        m1 = jnp.max(masked, axis=1, keepdims=True)
        i1 = jnp.min(jnp.where(masked >= m1, lane, 2**30), axis=1, keepdims=True)
        e1 = jnp.exp(m1 - m0)
        den = 1.0 + e1
        s0_ref[...] = 1.0 / den
        s1_ref[...] = e1 / den
        i0_ref[...] = i0
        i1_ref[...] = i1
        o_ref[...] = jnp.zeros_like(o_ref)

    coef = (s0_ref[...] * (i0_ref[...] == e) +
            s1_ref[...] * (i1_ref[...] == e))

    hb = jnp.dot(x_ref[...], w1_ref[0], preferred_element_type=jnp.float32)
    hb = jnp.maximum(hb + b1_ref[0], 0.0)
    contrib = jnp.dot(hb, w2_ref[0], preferred_element_type=jnp.float32)
    o_ref[...] += coef * contrib

    @pl.when(h == 0)
    def _bias2():
        o_ref[...] += coef * b2_ref[0]


def kernel(x, Wg, bg, W1, b1, W2, b2):
    x2 = x.reshape(SEQ, D_MODEL)
    wgp = jnp.pad(Wg, ((0, 0), (0, 128 - NUM_EXPERTS)))
    bgp = jnp.pad(bg, (0, 128 - NUM_EXPERTS), constant_values=NEG).reshape(1, 128)
    b1r = b1.reshape(NUM_EXPERTS, 1, HIDDEN)
    b2r = b2.reshape(NUM_EXPERTS, 1, D_MODEL)

    grid = (SEQ // TBLK, NUM_EXPERTS, H2)
    out = pl.pallas_call(
        _moe_body,
        grid=grid,
        in_specs=[
            pl.BlockSpec((TBLK, D_MODEL), lambda t, e, h: (t, 0)),
            pl.BlockSpec((D_MODEL, 128), lambda t, e, h: (0, 0)),
            pl.BlockSpec((1, 128), lambda t, e, h: (0, 0)),
            pl.BlockSpec((1, D_MODEL, HBLK), lambda t, e, h: (e, 0, h)),
            pl.BlockSpec((1, 1, HBLK), lambda t, e, h: (e, 0, h)),
            pl.BlockSpec((1, HBLK, D_MODEL), lambda t, e, h: (e, h, 0)),
            pl.BlockSpec((1, 1, D_MODEL), lambda t, e, h: (e, 0, 0)),
        ],
        out_specs=pl.BlockSpec((TBLK, D_MODEL), lambda t, e, h: (t, 0)),
        out_shape=jax.ShapeDtypeStruct((SEQ, D_MODEL), jnp.float32),
        scratch_shapes=[
            pltpu.VMEM((TBLK, 1), jnp.float32),
            pltpu.VMEM((TBLK, 1), jnp.float32),
            pltpu.VMEM((TBLK, 1), jnp.int32),
            pltpu.VMEM((TBLK, 1), jnp.int32),
        ],
    )(x2, wgp, bgp, W1, b1r, W2, b2r)
    return out.reshape(1, SEQ, D_MODEL)


# speedup vs baseline: 2.1946x; 2.1946x over previous
import functools

import jax
import jax.numpy as jnp
from jax import lax
from jax.experimental import pallas as pl
from jax.experimental.pallas import tpu as pltpu
from jax.experimental.pallas import tpu_sc as plsc

NUM_EXPERTS = 8
D_MODEL = 768
HIDDEN = 3072
SEQ = 2048
NASSIGN = 2 * SEQ
BLK = 128
CAP = NASSIGN + NUM_EXPERTS * BLK
NB = CAP // BLK
NEG = -1e30

NW = 32
JPW = NASSIGN // NW
TPW = SEQ // NW
NCHUNK = D_MODEL // 16


def _router_body(x_ref, wg_ref, bg_ref, i0_ref, i1_ref, w0_ref, w1_ref):
    logits = jnp.dot(x_ref[...], wg_ref[...], preferred_element_type=jnp.float32)
    logits = logits + bg_ref[...]
    lane = jax.lax.broadcasted_iota(jnp.int32, logits.shape, 1)
    m0 = jnp.max(logits, axis=1, keepdims=True)
    i0 = jnp.min(jnp.where(logits >= m0, lane, 2**30), axis=1, keepdims=True)
    masked = jnp.where(lane == i0, NEG, logits)
    m1 = jnp.max(masked, axis=1, keepdims=True)
    i1 = jnp.min(jnp.where(masked >= m1, lane, 2**30), axis=1, keepdims=True)
    e1 = jnp.exp(m1 - m0)
    den = 1.0 + e1
    i0_ref[...] = i0
    i1_ref[...] = i1
    w0_ref[...] = jnp.broadcast_to(1.0 / den, (SEQ, 16))
    w1_ref[...] = jnp.broadcast_to(e1 / den, (SEQ, 16))


def _router(x2, wgp, bgp):
    return pl.pallas_call(
        _router_body,
        out_shape=(
            jax.ShapeDtypeStruct((SEQ, 1), jnp.int32),
            jax.ShapeDtypeStruct((SEQ, 1), jnp.int32),
            jax.ShapeDtypeStruct((SEQ, 16), jnp.float32),
            jax.ShapeDtypeStruct((SEQ, 16), jnp.float32),
        ),
    )(x2, wgp, bgp)


def _dispatch_body(e_ref, pos_ref, be_ref):
    ea = e_ref[...]
    r128 = jax.lax.broadcasted_iota(jnp.int32, (128, 128), 0)
    c128 = jax.lax.broadcasted_iota(jnp.int32, (128, 128), 1)
    ustrict = (r128 < c128).astype(jnp.float32)
    r32 = jax.lax.broadcasted_iota(jnp.int32, (32, 32), 0)
    c32 = jax.lax.broadcasted_iota(jnp.int32, (32, 32), 1)
    lstrict = (c32 < r32).astype(jnp.float32)

    bi = (jax.lax.broadcasted_iota(jnp.int32, (1, 128), 1) * BLK).astype(jnp.float32)

    pos = jnp.zeros((32, 128), jnp.float32)
    bef = jnp.zeros((1, 128), jnp.float32)
    start = jnp.float32(0.0)
    e_last = jnp.float32(0.0)
    for e in range(NUM_EXPERTS):
        onehot = (ea == e).astype(jnp.float32)
        prefix = jnp.dot(onehot, ustrict, preferred_element_type=jnp.float32)
        rowsum = jnp.sum(onehot, axis=1, keepdims=True)
        rowpfx = jnp.dot(lstrict, rowsum, preferred_element_type=jnp.float32)
        rank = prefix + rowpfx
        cnt = jnp.sum(rowsum)
        padded = jnp.ceil(cnt / BLK) * BLK
        pos = pos + onehot * (start + rank)
        ind = jnp.logical_and(bi >= start, bi < start + padded)
        bef = bef + e * ind.astype(jnp.float32)
        e_last = jnp.where(cnt > 0, jnp.float32(e), e_last)
        start = start + padded
    bef = jnp.where(bi >= start, e_last, bef)
    pos_ref[...] = pos.astype(jnp.int32)
    be_ref[...] = bef.astype(jnp.int32)


def _dispatch(e_all):
    return pl.pallas_call(
        _dispatch_body,
        out_shape=(
            jax.ShapeDtypeStruct((32, 128), jnp.int32),
            jax.ShapeDtypeStruct((1, 128), jnp.int32),
        ),
    )(e_all)


def _scatter_body(x_hbm, pos_hbm, xg_hbm, idx_v, rows_v, sem):
    wid = lax.axis_index("s") * 2 + lax.axis_index("c")
    k = wid // 16
    tbase = (wid % 16) * JPW
    jbase = k * SEQ + tbase
    pltpu.sync_copy(x_hbm.at[pl.ds(tbase, JPW)], rows_v)
    pltpu.sync_copy(pos_hbm.at[pl.ds(jbase, JPW)], idx_v)
    pltpu.async_copy(rows_v, xg_hbm.at[idx_v], sem).wait()


@functools.cache
def _scatter():
    return pl.kernel(
        _scatter_body,
        out_type=jax.ShapeDtypeStruct((CAP, D_MODEL), jnp.float32),
        mesh=plsc.VectorSubcoreMesh(core_axis_name="c", subcore_axis_name="s"),
        scratch_types=[
            pltpu.VMEM((JPW,), jnp.int32),
            pltpu.VMEM((JPW, D_MODEL), jnp.float32),
            pltpu.SemaphoreType.DMA,
        ],
    )


def _ffn_body(be_ref, x_ref, w1_ref, b1_ref, w2_ref, b2_ref, o_ref):
    h = jnp.dot(x_ref[...], w1_ref[0], preferred_element_type=jnp.float32)
    h = jnp.maximum(h + b1_ref[0], 0.0)
    o = jnp.dot(h, w2_ref[0], preferred_element_type=jnp.float32)
    o_ref[...] = o + b2_ref[0]


def _ffn(be, xg, W1, b1r, W2, b2r):
    grid_spec = pltpu.PrefetchScalarGridSpec(
        num_scalar_prefetch=1,
        grid=(NB,),
        in_specs=[
            pl.BlockSpec((BLK, D_MODEL), lambda b, be: (b, 0)),
            pl.BlockSpec((1, D_MODEL, HIDDEN), lambda b, be: (be[b], 0, 0)),
            pl.BlockSpec((1, 1, HIDDEN), lambda b, be: (be[b], 0, 0)),
            pl.BlockSpec((1, HIDDEN, D_MODEL), lambda b, be: (be[b], 0, 0)),
            pl.BlockSpec((1, 1, D_MODEL), lambda b, be: (be[b], 0, 0)),
        ],
        out_specs=pl.BlockSpec((BLK, D_MODEL), lambda b, be: (b, 0)),
    )
    return pl.pallas_call(
        _ffn_body,
        grid_spec=grid_spec,
        out_shape=jax.ShapeDtypeStruct((CAP, D_MODEL), jnp.float32),
    )(be, xg, W1, b1r, W2, b2r)


def _combine_body(yg_hbm, pos_hbm, w0_hbm, w1_hbm, out_hbm,
                  p0_v, p1_v, w0_v, w1_v, y0_v, y1_v, sem):
    wid = lax.axis_index("s") * 2 + lax.axis_index("c")
    base = wid * TPW
    pltpu.sync_copy(pos_hbm.at[pl.ds(base, TPW)], p0_v)
    pltpu.sync_copy(pos_hbm.at[pl.ds(SEQ + base, TPW)], p1_v)
    pltpu.async_copy(yg_hbm.at[p0_v], y0_v, sem).wait()
    pltpu.async_copy(yg_hbm.at[p1_v], y1_v, sem).wait()
    pltpu.sync_copy(w0_hbm.at[pl.ds(base, TPW)], w0_v)
    pltpu.sync_copy(w1_hbm.at[pl.ds(base, TPW)], w1_v)

    def _token(i, _):
        s0 = w0_v[i]
        s1 = w1_v[i]
        for c in range(NCHUNK):
            sl = pl.ds(c * 16, 16)
            y0_v[i, sl] = y0_v[i, sl] * s0 + y1_v[i, sl] * s1
        return 0

    lax.fori_loop(0, TPW, _token, 0)
    pltpu.sync_copy(y0_v, out_hbm.at[pl.ds(base, TPW)])


@functools.cache
def _combine():
    return pl.kernel(
        _combine_body,
        out_type=jax.ShapeDtypeStruct((SEQ, D_MODEL), jnp.float32),
        mesh=plsc.VectorSubcoreMesh(core_axis_name="c", subcore_axis_name="s"),
        scratch_types=[
            pltpu.VMEM((TPW,), jnp.int32),
            pltpu.VMEM((TPW,), jnp.int32),
            pltpu.VMEM((TPW, 16), jnp.float32),
            pltpu.VMEM((TPW, 16), jnp.float32),
            pltpu.VMEM((TPW, D_MODEL), jnp.float32),
            pltpu.VMEM((TPW, D_MODEL), jnp.float32),
            pltpu.SemaphoreType.DMA,
        ],
    )


def kernel(x, Wg, bg, W1, b1, W2, b2):
    x2 = x.reshape(SEQ, D_MODEL)
    wgp = jnp.pad(Wg, ((0, 0), (0, 128 - NUM_EXPERTS)))
    bgp = jnp.pad(bg, (0, 128 - NUM_EXPERTS), constant_values=NEG).reshape(1, 128)
    b1r = b1.reshape(NUM_EXPERTS, 1, HIDDEN)
    b2r = b2.reshape(NUM_EXPERTS, 1, D_MODEL)

    i0, i1, w0, w1 = _router(x2, wgp, bgp)
    e_all = jnp.concatenate(
        [i0.reshape(16, 128), i1.reshape(16, 128)], axis=0)
    pos, be = _dispatch(e_all)
    posflat = pos.reshape(NASSIGN)
    be_flat = be.reshape(128)[:NB]

    xg = _scatter()(x2, posflat)
    yg = _ffn(be_flat, xg, W1, b1r, W2, b2r)
    out = _combine()(yg, posflat, w0, w1)
    return out.reshape(1, SEQ, D_MODEL)


# speedup vs baseline: 2.2891x; 1.0431x over previous
import functools

import jax
import jax.numpy as jnp
from jax import lax
from jax.experimental import pallas as pl
from jax.experimental.pallas import tpu as pltpu
from jax.experimental.pallas import tpu_sc as plsc

NUM_EXPERTS = 8
D_MODEL = 768
HIDDEN = 3072
SEQ = 2048
NASSIGN = 2 * SEQ
BLK = 256
CAP = NASSIGN + NUM_EXPERTS * BLK
NB = CAP // BLK
NEG = -1e30

NW = 32
JPW = NASSIGN // NW
TPW = SEQ // NW
NCHUNK = D_MODEL // 16


def _router_body(x_ref, wg_ref, bg_ref, i0_ref, i1_ref, w0_ref, w1_ref):
    logits = jnp.dot(x_ref[...], wg_ref[...], preferred_element_type=jnp.float32)
    logits = logits + bg_ref[...]
    lane = jax.lax.broadcasted_iota(jnp.int32, logits.shape, 1)
    m0 = jnp.max(logits, axis=1, keepdims=True)
    i0 = jnp.min(jnp.where(logits >= m0, lane, 2**30), axis=1, keepdims=True)
    masked = jnp.where(lane == i0, NEG, logits)
    m1 = jnp.max(masked, axis=1, keepdims=True)
    i1 = jnp.min(jnp.where(masked >= m1, lane, 2**30), axis=1, keepdims=True)
    e1 = jnp.exp(m1 - m0)
    den = 1.0 + e1
    i0_ref[...] = i0
    i1_ref[...] = i1
    w0_ref[...] = jnp.broadcast_to(1.0 / den, (SEQ, 16))
    w1_ref[...] = jnp.broadcast_to(e1 / den, (SEQ, 16))


def _router(x2, wgp, bgp):
    return pl.pallas_call(
        _router_body,
        out_shape=(
            jax.ShapeDtypeStruct((SEQ, 1), jnp.int32),
            jax.ShapeDtypeStruct((SEQ, 1), jnp.int32),
            jax.ShapeDtypeStruct((SEQ, 16), jnp.float32),
            jax.ShapeDtypeStruct((SEQ, 16), jnp.float32),
        ),
    )(x2, wgp, bgp)


def _dispatch_body(e_ref, pos_ref, be_ref):
    ea = e_ref[...]
    r128 = jax.lax.broadcasted_iota(jnp.int32, (128, 128), 0)
    c128 = jax.lax.broadcasted_iota(jnp.int32, (128, 128), 1)
    ustrict = (r128 < c128).astype(jnp.float32)
    r32 = jax.lax.broadcasted_iota(jnp.int32, (32, 32), 0)
    c32 = jax.lax.broadcasted_iota(jnp.int32, (32, 32), 1)
    lstrict = (c32 < r32).astype(jnp.float32)

    bi = (jax.lax.broadcasted_iota(jnp.int32, (1, 128), 1) * BLK).astype(jnp.float32)

    pos = jnp.zeros((32, 128), jnp.float32)
    bef = jnp.zeros((1, 128), jnp.float32)
    start = jnp.float32(0.0)
    e_last = jnp.float32(0.0)
    for e in range(NUM_EXPERTS):
        onehot = (ea == e).astype(jnp.float32)
        prefix = jnp.dot(onehot, ustrict, preferred_element_type=jnp.float32)
        rowsum = jnp.sum(onehot, axis=1, keepdims=True)
        rowpfx = jnp.dot(lstrict, rowsum, preferred_element_type=jnp.float32)
        rank = prefix + rowpfx
        cnt = jnp.sum(rowsum)
        padded = jnp.ceil(cnt / BLK) * BLK
        pos = pos + onehot * (start + rank)
        ind = jnp.logical_and(bi >= start, bi < start + padded)
        bef = bef + e * ind.astype(jnp.float32)
        e_last = jnp.where(cnt > 0, jnp.float32(e), e_last)
        start = start + padded
    bef = jnp.where(bi >= start, e_last, bef)
    pos_ref[...] = pos.astype(jnp.int32)
    be_ref[...] = bef.astype(jnp.int32)


def _dispatch(e_all):
    return pl.pallas_call(
        _dispatch_body,
        out_shape=(
            jax.ShapeDtypeStruct((32, 128), jnp.int32),
            jax.ShapeDtypeStruct((1, 128), jnp.int32),
        ),
    )(e_all)


def _scatter_body(x_hbm, pos_hbm, xg_hbm, idx_v, rows_v, sem):
    wid = lax.axis_index("s") * 2 + lax.axis_index("c")
    k = wid // 16
    tbase = (wid % 16) * JPW
    jbase = k * SEQ + tbase
    pltpu.sync_copy(x_hbm.at[pl.ds(tbase, JPW)], rows_v)
    pltpu.sync_copy(pos_hbm.at[pl.ds(jbase, JPW)], idx_v)
    pltpu.async_copy(rows_v, xg_hbm.at[idx_v], sem).wait()


@functools.cache
def _scatter():
    return pl.kernel(
        _scatter_body,
        out_type=jax.ShapeDtypeStruct((CAP, D_MODEL), jnp.float32),
        mesh=plsc.VectorSubcoreMesh(core_axis_name="c", subcore_axis_name="s"),
        scratch_types=[
            pltpu.VMEM((JPW,), jnp.int32),
            pltpu.VMEM((JPW, D_MODEL), jnp.float32),
            pltpu.SemaphoreType.DMA,
        ],
    )


def _ffn_body(be_ref, x_ref, w1_ref, b1_ref, w2_ref, b2_ref, o_ref):
    xb = x_ref[...].astype(jnp.bfloat16)
    h = jnp.dot(xb, w1_ref[0].astype(jnp.bfloat16),
                preferred_element_type=jnp.float32)
    h = jnp.maximum(h + b1_ref[0], 0.0)
    o = jnp.dot(h.astype(jnp.bfloat16), w2_ref[0].astype(jnp.bfloat16),
                preferred_element_type=jnp.float32)
    o_ref[...] = o + b2_ref[0]


def _ffn(be, xg, W1, b1r, W2, b2r):
    grid_spec = pltpu.PrefetchScalarGridSpec(
        num_scalar_prefetch=1,
        grid=(NB,),
        in_specs=[
            pl.BlockSpec((BLK, D_MODEL), lambda b, be: (b, 0)),
            pl.BlockSpec((1, D_MODEL, HIDDEN), lambda b, be: (be[b], 0, 0)),
            pl.BlockSpec((1, 1, HIDDEN), lambda b, be: (be[b], 0, 0)),
            pl.BlockSpec((1, HIDDEN, D_MODEL), lambda b, be: (be[b], 0, 0)),
            pl.BlockSpec((1, 1, D_MODEL), lambda b, be: (be[b], 0, 0)),
        ],
        out_specs=pl.BlockSpec((BLK, D_MODEL), lambda b, be: (b, 0)),
    )
    return pl.pallas_call(
        _ffn_body,
        grid_spec=grid_spec,
        out_shape=jax.ShapeDtypeStruct((CAP, D_MODEL), jnp.float32),
    )(be, xg, W1, b1r, W2, b2r)


def _combine_body(yg_hbm, pos_hbm, w0_hbm, w1_hbm, out_hbm,
                  p0_v, p1_v, w0_v, w1_v, y0_v, y1_v, sem):
    wid = lax.axis_index("s") * 2 + lax.axis_index("c")
    base = wid * TPW
    pltpu.sync_copy(pos_hbm.at[pl.ds(base, TPW)], p0_v)
    pltpu.sync_copy(pos_hbm.at[pl.ds(SEQ + base, TPW)], p1_v)
    pltpu.async_copy(yg_hbm.at[p0_v], y0_v, sem).wait()
    pltpu.async_copy(yg_hbm.at[p1_v], y1_v, sem).wait()
    pltpu.sync_copy(w0_hbm.at[pl.ds(base, TPW)], w0_v)
    pltpu.sync_copy(w1_hbm.at[pl.ds(base, TPW)], w1_v)

    def _token(i, _):
        s0 = w0_v[i]
        s1 = w1_v[i]
        for c in range(NCHUNK):
            sl = pl.ds(c * 16, 16)
            y0_v[i, sl] = y0_v[i, sl] * s0 + y1_v[i, sl] * s1
        return 0

    lax.fori_loop(0, TPW, _token, 0)
    pltpu.sync_copy(y0_v, out_hbm.at[pl.ds(base, TPW)])


@functools.cache
def _combine():
    return pl.kernel(
        _combine_body,
        out_type=jax.ShapeDtypeStruct((SEQ, D_MODEL), jnp.float32),
        mesh=plsc.VectorSubcoreMesh(core_axis_name="c", subcore_axis_name="s"),
        scratch_types=[
            pltpu.VMEM((TPW,), jnp.int32),
            pltpu.VMEM((TPW,), jnp.int32),
            pltpu.VMEM((TPW, 16), jnp.float32),
            pltpu.VMEM((TPW, 16), jnp.float32),
            pltpu.VMEM((TPW, D_MODEL), jnp.float32),
            pltpu.VMEM((TPW, D_MODEL), jnp.float32),
            pltpu.SemaphoreType.DMA,
        ],
    )


def kernel(x, Wg, bg, W1, b1, W2, b2):
    x2 = x.reshape(SEQ, D_MODEL)
    wgp = jnp.pad(Wg, ((0, 0), (0, 128 - NUM_EXPERTS)))
    bgp = jnp.pad(bg, (0, 128 - NUM_EXPERTS), constant_values=NEG).reshape(1, 128)
    b1r = b1.reshape(NUM_EXPERTS, 1, HIDDEN)
    b2r = b2.reshape(NUM_EXPERTS, 1, D_MODEL)

    i0, i1, w0, w1 = _router(x2, wgp, bgp)
    e_all = jnp.concatenate(
        [i0.reshape(16, 128), i1.reshape(16, 128)], axis=0)
    pos, be = _dispatch(e_all)
    posflat = pos.reshape(NASSIGN)
    be_flat = be.reshape(128)[:NB]

    xg = _scatter()(x2, posflat)
    yg = _ffn(be_flat, xg, W1, b1r, W2, b2r)
    out = _combine()(yg, posflat, w0, w1)
    return out.reshape(1, SEQ, D_MODEL)


# speedup vs baseline: 2.3833x; 1.0412x over previous
import functools

import jax
import jax.numpy as jnp
from jax import lax
from jax.experimental import pallas as pl
from jax.experimental.pallas import tpu as pltpu
from jax.experimental.pallas import tpu_sc as plsc

NUM_EXPERTS = 8
D_MODEL = 768
HIDDEN = 3072
SEQ = 2048
NASSIGN = 2 * SEQ
BLK = 256
CAP = NASSIGN + NUM_EXPERTS * BLK
NB = CAP // BLK
NEG = -1e30

NW = 32
JPW = NASSIGN // NW
TPW = SEQ // NW
NCHUNK = D_MODEL // 16


def _route_body(x_ref, wg_ref, bg_ref, pos_ref, be_ref, w0_ref, w1_ref):
    logits = jnp.dot(x_ref[...], wg_ref[...], preferred_element_type=jnp.float32)
    logits = logits + bg_ref[...]
    lane = jax.lax.broadcasted_iota(jnp.int32, logits.shape, 1)
    m0 = jnp.max(logits, axis=1, keepdims=True)
    i0 = jnp.min(jnp.where(logits >= m0, lane, 2**30), axis=1, keepdims=True)
    masked = jnp.where(lane == i0, NEG, logits)
    m1 = jnp.max(masked, axis=1, keepdims=True)
    i1 = jnp.min(jnp.where(masked >= m1, lane, 2**30), axis=1, keepdims=True)
    e1 = jnp.exp(m1 - m0)
    den = 1.0 + e1
    w0_ref[...] = jnp.broadcast_to(1.0 / den, (SEQ, 16))
    w1_ref[...] = jnp.broadcast_to(e1 / den, (SEQ, 16))

    ea = jnp.concatenate(
        [jnp.reshape(i0, (16, 128)), jnp.reshape(i1, (16, 128))], axis=0)

    r128 = jax.lax.broadcasted_iota(jnp.int32, (128, 128), 0)
    c128 = jax.lax.broadcasted_iota(jnp.int32, (128, 128), 1)
    ustrict = (r128 < c128).astype(jnp.float32)
    r32 = jax.lax.broadcasted_iota(jnp.int32, (32, 32), 0)
    c32 = jax.lax.broadcasted_iota(jnp.int32, (32, 32), 1)
    lstrict = (c32 < r32).astype(jnp.float32)
    bi = (jax.lax.broadcasted_iota(jnp.int32, (1, 128), 1) * BLK).astype(jnp.float32)

    pos = jnp.zeros((32, 128), jnp.float32)
    bef = jnp.zeros((1, 128), jnp.float32)
    start = jnp.float32(0.0)
    e_last = jnp.float32(0.0)
    for e in range(NUM_EXPERTS):
        onehot = (ea == e).astype(jnp.float32)
        prefix = jnp.dot(onehot, ustrict, preferred_element_type=jnp.float32)
        rowsum = jnp.sum(onehot, axis=1, keepdims=True)
        rowpfx = jnp.dot(lstrict, rowsum, preferred_element_type=jnp.float32)
        rank = prefix + rowpfx
        cnt = jnp.sum(rowsum)
        padded = jnp.ceil(cnt / BLK) * BLK
        pos = pos + onehot * (start + rank)
        ind = jnp.logical_and(bi >= start, bi < start + padded)
        bef = bef + e * ind.astype(jnp.float32)
        e_last = jnp.where(cnt > 0, jnp.float32(e), e_last)
        start = start + padded
    bef = jnp.where(bi >= start, e_last, bef)
    pos_ref[...] = pos.astype(jnp.int32)
    be_ref[...] = bef.astype(jnp.int32)


def _route(x2, Wg, bg):
    return pl.pallas_call(
        _route_body,
        out_shape=(
            jax.ShapeDtypeStruct((32, 128), jnp.int32),
            jax.ShapeDtypeStruct((1, 128), jnp.int32),
            jax.ShapeDtypeStruct((SEQ, 16), jnp.float32),
            jax.ShapeDtypeStruct((SEQ, 16), jnp.float32),
        ),
    )(x2, Wg, bg)


def _scatter_body(x_hbm, pos_hbm, xg_hbm, idx_v, rows_v, sem):
    wid = lax.axis_index("s") * 2 + lax.axis_index("c")
    tbase = (wid % 16) * JPW
    pltpu.sync_copy(x_hbm.at[pl.ds(tbase, JPW)], rows_v)
    pltpu.sync_copy(pos_hbm.at[wid], idx_v)
    pltpu.async_copy(rows_v, xg_hbm.at[idx_v], sem).wait()


@functools.cache
def _scatter():
    return pl.kernel(
        _scatter_body,
        out_type=jax.ShapeDtypeStruct((CAP, D_MODEL), jnp.float32),
        mesh=plsc.VectorSubcoreMesh(core_axis_name="c", subcore_axis_name="s"),
        scratch_types=[
            pltpu.VMEM((JPW,), jnp.int32),
            pltpu.VMEM((JPW, D_MODEL), jnp.float32),
            pltpu.SemaphoreType.DMA,
        ],
    )


def _ffn_body(be_ref, x_ref, w1_ref, b1_ref, w2_ref, b2_ref, o_ref):
    xb = x_ref[...].astype(jnp.bfloat16)
    h = jnp.dot(xb, w1_ref[0].astype(jnp.bfloat16),
                preferred_element_type=jnp.float32)
    h = jnp.maximum(h + b1_ref[0], 0.0)
    o = jnp.dot(h.astype(jnp.bfloat16), w2_ref[0].astype(jnp.bfloat16),
                preferred_element_type=jnp.float32)
    o_ref[...] = o + b2_ref[0]


def _ffn(be, xg, W1, b1r, W2, b2r):
    grid_spec = pltpu.PrefetchScalarGridSpec(
        num_scalar_prefetch=1,
        grid=(NB,),
        in_specs=[
            pl.BlockSpec((BLK, D_MODEL), lambda b, be: (b, 0)),
            pl.BlockSpec((1, D_MODEL, HIDDEN), lambda b, be: (be[0, b], 0, 0)),
            pl.BlockSpec((1, 1, HIDDEN), lambda b, be: (be[0, b], 0, 0)),
            pl.BlockSpec((1, HIDDEN, D_MODEL), lambda b, be: (be[0, b], 0, 0)),
            pl.BlockSpec((1, 1, D_MODEL), lambda b, be: (be[0, b], 0, 0)),
        ],
        out_specs=pl.BlockSpec((BLK, D_MODEL), lambda b, be: (b, 0)),
    )
    return pl.pallas_call(
        _ffn_body,
        grid_spec=grid_spec,
        out_shape=jax.ShapeDtypeStruct((CAP, D_MODEL), jnp.float32),
    )(be, xg, W1, b1r, W2, b2r)


def _combine_body(yg_hbm, pos_hbm, w0_hbm, w1_hbm, out_hbm,
                  p0_v, p1_v, w0_v, w1_v, y0_v, y1_v, sem):
    wid = lax.axis_index("s") * 2 + lax.axis_index("c")
    base = wid * TPW
    row = wid // 2
    col = (wid % 2) * TPW
    pltpu.sync_copy(pos_hbm.at[row, pl.ds(col, TPW)], p0_v)
    pltpu.sync_copy(pos_hbm.at[16 + row, pl.ds(col, TPW)], p1_v)
    pltpu.async_copy(yg_hbm.at[p0_v], y0_v, sem).wait()
    pltpu.async_copy(yg_hbm.at[p1_v], y1_v, sem).wait()
    pltpu.sync_copy(w0_hbm.at[pl.ds(base, TPW)], w0_v)
    pltpu.sync_copy(w1_hbm.at[pl.ds(base, TPW)], w1_v)

    def _token(i, _):
        s0 = w0_v[i]
        s1 = w1_v[i]
        for c in range(NCHUNK):
            sl = pl.ds(c * 16, 16)
            y0_v[i, sl] = y0_v[i, sl] * s0 + y1_v[i, sl] * s1
        return 0

    lax.fori_loop(0, TPW, _token, 0)
    pltpu.sync_copy(y0_v, out_hbm.at[pl.ds(base, TPW)])


@functools.cache
def _combine():
    return pl.kernel(
        _combine_body,
        out_type=jax.ShapeDtypeStruct((SEQ, D_MODEL), jnp.float32),
        mesh=plsc.VectorSubcoreMesh(core_axis_name="c", subcore_axis_name="s"),
        scratch_types=[
            pltpu.VMEM((TPW,), jnp.int32),
            pltpu.VMEM((TPW,), jnp.int32),
            pltpu.VMEM((TPW, 16), jnp.float32),
            pltpu.VMEM((TPW, 16), jnp.float32),
            pltpu.VMEM((TPW, D_MODEL), jnp.float32),
            pltpu.VMEM((TPW, D_MODEL), jnp.float32),
            pltpu.SemaphoreType.DMA,
        ],
    )


def kernel(x, Wg, bg, W1, b1, W2, b2):
    x2 = x.reshape(SEQ, D_MODEL)
    b1r = b1.reshape(NUM_EXPERTS, 1, HIDDEN)
    b2r = b2.reshape(NUM_EXPERTS, 1, D_MODEL)

    pos, be, w0, w1 = _route(x2, Wg, bg)
    xg = _scatter()(x2, pos)
    yg = _ffn(be, xg, W1, b1r, W2, b2r)
    out = _combine()(yg, pos, w0, w1)
    return out.reshape(1, SEQ, D_MODEL)


# speedup vs baseline: 2.4207x; 1.0157x over previous
import functools

import jax
import jax.numpy as jnp
from jax import lax
from jax.experimental import pallas as pl
from jax.experimental.pallas import tpu as pltpu
from jax.experimental.pallas import tpu_sc as plsc

NUM_EXPERTS = 8
D_MODEL = 768
HIDDEN = 3072
SEQ = 2048
NASSIGN = 2 * SEQ
BLK = 256
CAP = NASSIGN + NUM_EXPERTS * BLK
NB = CAP // BLK
NEG = -1e30

NW = 32
JPW = NASSIGN // NW
TPW = SEQ // NW
NCHUNK = D_MODEL // 16


def _route_body(x_ref, wg_ref, bg_ref, pos_ref, be_ref, w0_ref, w1_ref):
    logits = jnp.dot(x_ref[...], wg_ref[...], preferred_element_type=jnp.float32)
    logits = logits + bg_ref[...]
    lane = jax.lax.broadcasted_iota(jnp.int32, logits.shape, 1)
    m0 = jnp.max(logits, axis=1, keepdims=True)
    i0 = jnp.min(jnp.where(logits >= m0, lane, 2**30), axis=1, keepdims=True)
    masked = jnp.where(lane == i0, NEG, logits)
    m1 = jnp.max(masked, axis=1, keepdims=True)
    i1 = jnp.min(jnp.where(masked >= m1, lane, 2**30), axis=1, keepdims=True)
    e1 = jnp.exp(m1 - m0)
    den = 1.0 + e1
    w0_ref[...] = jnp.broadcast_to(1.0 / den, (SEQ, 16))
    w1_ref[...] = jnp.broadcast_to(e1 / den, (SEQ, 16))

    ea = jnp.concatenate(
        [jnp.reshape(i0, (16, 128)), jnp.reshape(i1, (16, 128))], axis=0)

    r128 = jax.lax.broadcasted_iota(jnp.int32, (128, 128), 0)
    c128 = jax.lax.broadcasted_iota(jnp.int32, (128, 128), 1)
    ustrict = (r128 < c128).astype(jnp.float32)
    r32 = jax.lax.broadcasted_iota(jnp.int32, (32, 32), 0)
    c32 = jax.lax.broadcasted_iota(jnp.int32, (32, 32), 1)
    lstrict = (c32 < r32).astype(jnp.float32)
    bi = (jax.lax.broadcasted_iota(jnp.int32, (1, 128), 1) * BLK).astype(jnp.float32)

    pos = jnp.zeros((32, 128), jnp.float32)
    bef = jnp.zeros((1, 128), jnp.float32)
    start = jnp.float32(0.0)
    e_last = jnp.float32(0.0)
    for e in range(NUM_EXPERTS):
        onehot = (ea == e).astype(jnp.float32)
        prefix = jnp.dot(onehot, ustrict, preferred_element_type=jnp.float32)
        rowsum = jnp.sum(onehot, axis=1, keepdims=True)
        rowpfx = jnp.dot(lstrict, rowsum, preferred_element_type=jnp.float32)
        rank = prefix + rowpfx
        cnt = jnp.sum(rowsum)
        padded = jnp.ceil(cnt / BLK) * BLK
        pos = pos + onehot * (start + rank)
        ind = jnp.logical_and(bi >= start, bi < start + padded)
        bef = bef + e * ind.astype(jnp.float32)
        e_last = jnp.where(cnt > 0, jnp.float32(e), e_last)
        start = start + padded
    bef = jnp.where(bi >= start, e_last, bef)
    pos_ref[...] = pos.astype(jnp.int32)
    be_ref[...] = bef.astype(jnp.int32)


def _route(x2, Wg, bg):
    return pl.pallas_call(
        _route_body,
        out_shape=(
            jax.ShapeDtypeStruct((32, 128), jnp.int32),
            jax.ShapeDtypeStruct((1, 128), jnp.int32),
            jax.ShapeDtypeStruct((SEQ, 16), jnp.float32),
            jax.ShapeDtypeStruct((SEQ, 16), jnp.float32),
        ),
    )(x2, Wg, bg)


def _scatter_body(x_hbm, pos_hbm, xg_hbm, idx_v, rows_v, sem, sem2):
    wid = lax.axis_index("s") * 2 + lax.axis_index("c")
    tbase = (wid % 16) * JPW
    c0 = pltpu.async_copy(x_hbm.at[pl.ds(tbase, JPW)], rows_v, sem)
    c1 = pltpu.async_copy(pos_hbm.at[wid], idx_v, sem2)
    c0.wait()
    c1.wait()
    pltpu.async_copy(rows_v, xg_hbm.at[idx_v], sem).wait()


@functools.cache
def _scatter():
    return pl.kernel(
        _scatter_body,
        out_type=jax.ShapeDtypeStruct((CAP, D_MODEL), jnp.float32),
        mesh=plsc.VectorSubcoreMesh(core_axis_name="c", subcore_axis_name="s"),
        scratch_types=[
            pltpu.VMEM((JPW,), jnp.int32),
            pltpu.VMEM((JPW, D_MODEL), jnp.float32),
            pltpu.SemaphoreType.DMA,
            pltpu.SemaphoreType.DMA,
        ],
    )


def _ffn_body(be_ref, x_ref, w1_ref, b1_ref, w2_ref, b2_ref, o_ref):
    xb = x_ref[...].astype(jnp.bfloat16)
    h = jnp.dot(xb, w1_ref[0].astype(jnp.bfloat16),
                preferred_element_type=jnp.float32)
    h = jnp.maximum(h + b1_ref[0], 0.0)
    o = jnp.dot(h.astype(jnp.bfloat16), w2_ref[0].astype(jnp.bfloat16),
                preferred_element_type=jnp.float32)
    o_ref[...] = o + b2_ref[0]


def _ffn(be, xg, W1, b1r, W2, b2r):
    grid_spec = pltpu.PrefetchScalarGridSpec(
        num_scalar_prefetch=1,
        grid=(NB,),
        in_specs=[
            pl.BlockSpec((BLK, D_MODEL), lambda b, be: (b, 0)),
            pl.BlockSpec((1, D_MODEL, HIDDEN), lambda b, be: (be[0, b], 0, 0)),
            pl.BlockSpec((1, 1, HIDDEN), lambda b, be: (be[0, b], 0, 0)),
            pl.BlockSpec((1, HIDDEN, D_MODEL), lambda b, be: (be[0, b], 0, 0)),
            pl.BlockSpec((1, 1, D_MODEL), lambda b, be: (be[0, b], 0, 0)),
        ],
        out_specs=pl.BlockSpec((BLK, D_MODEL), lambda b, be: (b, 0)),
    )
    return pl.pallas_call(
        _ffn_body,
        grid_spec=grid_spec,
        out_shape=jax.ShapeDtypeStruct((CAP, D_MODEL), jnp.float32),
    )(be, xg, W1, b1r, W2, b2r)


def _combine_body(yg_hbm, pos_hbm, w0_hbm, w1_hbm, out_hbm,
                  p0_v, p1_v, w0_v, w1_v, y0_v, y1_v, sem):
    wid = lax.axis_index("s") * 2 + lax.axis_index("c")
    base = wid * TPW
    row = wid // 2
    col = (wid % 2) * TPW
    pltpu.sync_copy(pos_hbm.at[row, pl.ds(col, TPW)], p0_v)
    pltpu.sync_copy(pos_hbm.at[16 + row, pl.ds(col, TPW)], p1_v)
    g0 = pltpu.async_copy(yg_hbm.at[p0_v], y0_v, sem)
    g1 = pltpu.async_copy(yg_hbm.at[p1_v], y1_v, sem)
    pltpu.sync_copy(w0_hbm.at[pl.ds(base, TPW)], w0_v)
    pltpu.sync_copy(w1_hbm.at[pl.ds(base, TPW)], w1_v)
    g0.wait()
    g1.wait()

    def _token(i, _):
        s0 = w0_v[i]
        s1 = w1_v[i]
        for c in range(NCHUNK):
            sl = pl.ds(c * 16, 16)
            y0_v[i, sl] = y0_v[i, sl] * s0 + y1_v[i, sl] * s1
        return 0

    lax.fori_loop(0, TPW, _token, 0)
    pltpu.sync_copy(y0_v, out_hbm.at[pl.ds(base, TPW)])


@functools.cache
def _combine():
    return pl.kernel(
        _combine_body,
        out_type=jax.ShapeDtypeStruct((SEQ, D_MODEL), jnp.float32),
        mesh=plsc.VectorSubcoreMesh(core_axis_name="c", subcore_axis_name="s"),
        scratch_types=[
            pltpu.VMEM((TPW,), jnp.int32),
            pltpu.VMEM((TPW,), jnp.int32),
            pltpu.VMEM((TPW, 16), jnp.float32),
            pltpu.VMEM((TPW, 16), jnp.float32),
            pltpu.VMEM((TPW, D_MODEL), jnp.float32),
            pltpu.VMEM((TPW, D_MODEL), jnp.float32),
            pltpu.SemaphoreType.DMA,
        ],
    )


def kernel(x, Wg, bg, W1, b1, W2, b2):
    x2 = x.reshape(SEQ, D_MODEL)
    b1r = b1.reshape(NUM_EXPERTS, 1, HIDDEN)
    b2r = b2.reshape(NUM_EXPERTS, 1, D_MODEL)

    pos, be, w0, w1 = _route(x2, Wg, bg)
    xg = _scatter()(x2, pos)
    yg = _ffn(be, xg, W1, b1r, W2, b2r)
    out = _combine()(yg, pos, w0, w1)
    return out.reshape(1, SEQ, D_MODEL)


# speedup vs baseline: 2.4396x; 1.0078x over previous
import functools

import jax
import jax.numpy as jnp
from jax import lax
from jax.experimental import pallas as pl
from jax.experimental.pallas import tpu as pltpu
from jax.experimental.pallas import tpu_sc as plsc

NUM_EXPERTS = 8
D_MODEL = 768
HIDDEN = 3072
SEQ = 2048
NASSIGN = 2 * SEQ
BLK = 256
CAP = NASSIGN + NUM_EXPERTS * BLK
NB = CAP // BLK
NEG = -1e30

NW = 32
JPW = NASSIGN // NW
TPW = SEQ // NW
NCHUNK = D_MODEL // 16


def _route_body(x_ref, wg_ref, bg_ref, pos_ref, be_ref, w01_ref):
    logits = jnp.dot(x_ref[...], wg_ref[...], preferred_element_type=jnp.float32)
    logits = logits + bg_ref[...]
    lane = jax.lax.broadcasted_iota(jnp.int32, logits.shape, 1)
    m0 = jnp.max(logits, axis=1, keepdims=True)
    i0 = jnp.min(jnp.where(logits >= m0, lane, 2**30), axis=1, keepdims=True)
    masked = jnp.where(lane == i0, NEG, logits)
    m1 = jnp.max(masked, axis=1, keepdims=True)
    i1 = jnp.min(jnp.where(masked >= m1, lane, 2**30), axis=1, keepdims=True)
    e1 = jnp.exp(m1 - m0)
    den = 1.0 + e1
    w01_ref[...] = jnp.concatenate(
        [jnp.broadcast_to(1.0 / den, (SEQ, 16)),
         jnp.broadcast_to(e1 / den, (SEQ, 16))], axis=1)

    ea = jnp.concatenate(
        [jnp.reshape(i0, (16, 128)), jnp.reshape(i1, (16, 128))], axis=0)

    r128 = jax.lax.broadcasted_iota(jnp.int32, (128, 128), 0)
    c128 = jax.lax.broadcasted_iota(jnp.int32, (128, 128), 1)
    ustrict = (r128 < c128).astype(jnp.float32)
    r32 = jax.lax.broadcasted_iota(jnp.int32, (32, 32), 0)
    c32 = jax.lax.broadcasted_iota(jnp.int32, (32, 32), 1)
    lstrict = (c32 < r32).astype(jnp.float32)
    bi = (jax.lax.broadcasted_iota(jnp.int32, (1, 128), 1) * BLK).astype(jnp.float32)

    pos = jnp.zeros((32, 128), jnp.float32)
    bef = jnp.zeros((1, 128), jnp.float32)
    start = jnp.float32(0.0)
    e_last = jnp.float32(0.0)
    for e in range(NUM_EXPERTS):
        onehot = (ea == e).astype(jnp.float32)
        prefix = jnp.dot(onehot, ustrict, preferred_element_type=jnp.float32)
        rowsum = jnp.sum(onehot, axis=1, keepdims=True)
        rowpfx = jnp.dot(lstrict, rowsum, preferred_element_type=jnp.float32)
        rank = prefix + rowpfx
        cnt = jnp.sum(rowsum)
        padded = jnp.ceil(cnt / BLK) * BLK
        pos = pos + onehot * (start + rank)
        ind = jnp.logical_and(bi >= start, bi < start + padded)
        bef = bef + e * ind.astype(jnp.float32)
        e_last = jnp.where(cnt > 0, jnp.float32(e), e_last)
        start = start + padded
    bef = jnp.where(bi >= start, e_last, bef)
    pos_ref[...] = pos.astype(jnp.int32)
    be_ref[...] = bef.astype(jnp.int32)


def _route(x2, Wg, bg):
    return pl.pallas_call(
        _route_body,
        out_shape=(
            jax.ShapeDtypeStruct((32, 128), jnp.int32),
            jax.ShapeDtypeStruct((1, 128), jnp.int32),
            jax.ShapeDtypeStruct((SEQ, 32), jnp.float32),
        ),
    )(x2, Wg, bg)


def _scatter_body(x_hbm, pos_hbm, xg_hbm, idx_v, rows_v, sem, sem2):
    wid = lax.axis_index("s") * 2 + lax.axis_index("c")
    tbase = (wid % 16) * JPW
    c0 = pltpu.async_copy(x_hbm.at[pl.ds(tbase, JPW)], rows_v, sem)
    c1 = pltpu.async_copy(pos_hbm.at[wid], idx_v, sem2)
    c0.wait()
    c1.wait()
    pltpu.async_copy(rows_v, xg_hbm.at[idx_v], sem).wait()


@functools.cache
def _scatter():
    return pl.kernel(
        _scatter_body,
        out_type=jax.ShapeDtypeStruct((CAP, D_MODEL), jnp.float32),
        mesh=plsc.VectorSubcoreMesh(core_axis_name="c", subcore_axis_name="s"),
        scratch_types=[
            pltpu.VMEM((JPW,), jnp.int32),
            pltpu.VMEM((JPW, D_MODEL), jnp.float32),
            pltpu.SemaphoreType.DMA,
            pltpu.SemaphoreType.DMA,
        ],
    )


def _ffn_body(be_ref, x_ref, w1_ref, b1_ref, w2_ref, b2_ref, o_ref):
    b = pl.program_id(0)
    e = be_ref[0, b]
    erow0 = jax.lax.broadcasted_iota(jnp.int32, (NUM_EXPERTS, 1), 0) == e
    b1v = jnp.sum(jnp.where(erow0, b1_ref[...], 0.0), axis=0, keepdims=True)
    b2v = jnp.sum(jnp.where(erow0, b2_ref[...], 0.0), axis=0, keepdims=True)
    xb = x_ref[...].astype(jnp.bfloat16)
    h = jnp.dot(xb, w1_ref[0].astype(jnp.bfloat16),
                preferred_element_type=jnp.float32)
    h = jnp.maximum(h + b1v, 0.0)
    o = jnp.dot(h.astype(jnp.bfloat16), w2_ref[0].astype(jnp.bfloat16),
                preferred_element_type=jnp.float32)
    o_ref[...] = o + b2v


def _ffn(be, xg, W1, b1, W2, b2):
    grid_spec = pltpu.PrefetchScalarGridSpec(
        num_scalar_prefetch=1,
        grid=(NB,),
        in_specs=[
            pl.BlockSpec((BLK, D_MODEL), lambda b, be: (b, 0)),
            pl.BlockSpec((1, D_MODEL, HIDDEN), lambda b, be: (be[0, b], 0, 0)),
            pl.BlockSpec((NUM_EXPERTS, HIDDEN), lambda b, be: (0, 0)),
            pl.BlockSpec((1, HIDDEN, D_MODEL), lambda b, be: (be[0, b], 0, 0)),
            pl.BlockSpec((NUM_EXPERTS, D_MODEL), lambda b, be: (0, 0)),
        ],
        out_specs=pl.BlockSpec((BLK, D_MODEL), lambda b, be: (b, 0)),
    )
    return pl.pallas_call(
        _ffn_body,
        grid_spec=grid_spec,
        out_shape=jax.ShapeDtypeStruct((CAP, D_MODEL), jnp.float32),
    )(be, xg, W1, b1, W2, b2)


def _combine_body(yg_hbm, pos_hbm, w01_hbm, out_hbm,
                  p0_v, p1_v, w_v, y0_v, y1_v, sem):
    wid = lax.axis_index("s") * 2 + lax.axis_index("c")
    base = wid * TPW
    row = wid // 2
    col = (wid % 2) * TPW
    pltpu.sync_copy(pos_hbm.at[row, pl.ds(col, TPW)], p0_v)
    pltpu.sync_copy(pos_hbm.at[16 + row, pl.ds(col, TPW)], p1_v)
    g0 = pltpu.async_copy(yg_hbm.at[p0_v], y0_v, sem)
    g1 = pltpu.async_copy(yg_hbm.at[p1_v], y1_v, sem)
    pltpu.sync_copy(w01_hbm.at[pl.ds(base, TPW)], w_v)
    g0.wait()
    g1.wait()

    def _token(i, _):
        s0 = w_v[i, pl.ds(0, 16)]
        s1 = w_v[i, pl.ds(16, 16)]
        for c in range(NCHUNK):
            sl = pl.ds(c * 16, 16)
            y0_v[i, sl] = y0_v[i, sl] * s0 + y1_v[i, sl] * s1
        return 0

    lax.fori_loop(0, TPW, _token, 0)
    pltpu.sync_copy(y0_v, out_hbm.at[pl.ds(base, TPW)])


@functools.cache
def _combine():
    return pl.kernel(
        _combine_body,
        out_type=jax.ShapeDtypeStruct((SEQ, D_MODEL), jnp.float32),
        mesh=plsc.VectorSubcoreMesh(core_axis_name="c", subcore_axis_name="s"),
        scratch_types=[
            pltpu.VMEM((TPW,), jnp.int32),
            pltpu.VMEM((TPW,), jnp.int32),
            pltpu.VMEM((TPW, 32), jnp.float32),
            pltpu.VMEM((TPW, D_MODEL), jnp.float32),
            pltpu.VMEM((TPW, D_MODEL), jnp.float32),
            pltpu.SemaphoreType.DMA,
        ],
    )


def kernel(x, Wg, bg, W1, b1, W2, b2):
    x2 = x.reshape(SEQ, D_MODEL)

    pos, be, w01 = _route(x2, Wg, bg)
    xg = _scatter()(x2, pos)
    yg = _ffn(be, xg, W1, b1, W2, b2)
    out = _combine()(yg, pos, w01)
    return out.reshape(1, SEQ, D_MODEL)
